# Initial kernel scaffold; baseline (speedup 1.0000x reference)
import jax
import jax.numpy as jnp
from jax.experimental import pallas as pl


def kernel(positions, encoding_matrix):
    raise NotImplementedError("write your pallas kernel here")


# speedup vs baseline: 4.1112x; 4.1112x over previous
import functools

import jax
import jax.numpy as jnp
from jax import lax
from jax.experimental import pallas as pl
from jax.experimental.pallas import tpu as pltpu
from jax.experimental.pallas import tpu_sc as plsc

_D = 64
_CH = 128
_NW = 32


@functools.lru_cache(maxsize=None)
def _build(n_idx):
    per_w = n_idx // _NW
    n_chunks = per_w // _CH
    mesh = plsc.VectorSubcoreMesh(core_axis_name="c", subcore_axis_name="s")

    @functools.partial(
        pl.kernel,
        out_type=jax.ShapeDtypeStruct((n_idx, _D), jnp.float32),
        mesh=mesh,
        scratch_types=[
            pltpu.VMEM((per_w,), jnp.int32),
            pltpu.VMEM((_CH, _D), jnp.float32),
            pltpu.VMEM((_CH, _D), jnp.float32),
            pltpu.SemaphoreType.DMA,
            pltpu.SemaphoreType.DMA,
        ],
        compiler_params=pltpu.CompilerParams(use_tc_tiling_on_sc=False),
    )
    def gather_kernel(idx_hbm, table_hbm, out_hbm, idx_v, buf0, buf1, g0, g1):
        wid = lax.axis_index("s") * 2 + lax.axis_index("c")
        base = wid * per_w
        pltpu.sync_copy(idx_hbm.at[pl.ds(base, per_w)], idx_v)

        def issue(c, buf, sem):
            pltpu.async_copy(
                table_hbm.at[idx_v.at[pl.ds(c * _CH, _CH)]], buf, sem)

        def drain(c, buf, sem):
            pltpu.make_async_copy(
                table_hbm.at[idx_v.at[pl.ds(c * _CH, _CH)]], buf, sem).wait()
            pltpu.sync_copy(buf, out_hbm.at[pl.ds(base + c * _CH, _CH)])

        issue(0, buf0, g0)

        @pl.loop(0, n_chunks - 2, step=2)
        def _(c):
            issue(c + 1, buf1, g1)
            drain(c, buf0, g0)
            issue(c + 2, buf0, g0)
            drain(c + 1, buf1, g1)

        issue(n_chunks - 1, buf1, g1)
        drain(n_chunks - 2, buf0, g0)
        drain(n_chunks - 1, buf1, g1)

    return gather_kernel


def kernel(positions, encoding_matrix):
    lead_shape = positions.shape
    flat = positions.reshape(-1)
    out = _build(flat.size)(flat, encoding_matrix)
    return out.reshape(*lead_shape, _D)


# speedup vs baseline: 4.2625x; 1.0368x over previous
import functools

import jax
import jax.numpy as jnp
from jax import lax
from jax.experimental import pallas as pl
from jax.experimental.pallas import tpu as pltpu
from jax.experimental.pallas import tpu_sc as plsc

_D = 64
_CH = 128
_K = 4
_SC = _CH * _K
_NW = 32


@functools.lru_cache(maxsize=None)
def _build(n_idx):
    per_w = n_idx // _NW
    n_super = per_w // _SC
    mesh = plsc.VectorSubcoreMesh(core_axis_name="c", subcore_axis_name="s")

    @functools.partial(
        pl.kernel,
        out_type=jax.ShapeDtypeStruct((n_idx, _D), jnp.float32),
        mesh=mesh,
        scratch_types=[
            pltpu.VMEM((per_w,), jnp.int32),
            pltpu.VMEM((_SC, _D), jnp.float32),
            pltpu.VMEM((_SC, _D), jnp.float32),
            pltpu.SemaphoreType.DMA,
            pltpu.SemaphoreType.DMA,
        ],
        compiler_params=pltpu.CompilerParams(use_tc_tiling_on_sc=False),
    )
    def gather_kernel(idx_hbm, table_hbm, out_hbm, idx_v, buf0, buf1, g0, g1):
        wid = lax.axis_index("s") * 2 + lax.axis_index("c")
        base = wid * per_w
        pltpu.sync_copy(idx_hbm.at[pl.ds(base, per_w)], idx_v)

        def issue(s, buf, sem):
            for j in range(_K):
                pltpu.async_copy(
                    table_hbm.at[idx_v.at[pl.ds(s * _SC + j * _CH, _CH)]],
                    buf.at[pl.ds(j * _CH, _CH)], sem)

        def drain(s, buf, sem):
            for j in range(_K):
                pltpu.make_async_copy(
                    table_hbm.at[idx_v.at[pl.ds(s * _SC + j * _CH, _CH)]],
                    buf.at[pl.ds(j * _CH, _CH)], sem).wait()
            pltpu.sync_copy(buf, out_hbm.at[pl.ds(base + s * _SC, _SC)])

        issue(0, buf0, g0)

        @pl.loop(0, n_super - 2, step=2)
        def _(s):
            issue(s + 1, buf1, g1)
            drain(s, buf0, g0)
            issue(s + 2, buf0, g0)
            drain(s + 1, buf1, g1)

        issue(n_super - 1, buf1, g1)
        drain(n_super - 2, buf0, g0)
        drain(n_super - 1, buf1, g1)

    return gather_kernel


def kernel(positions, encoding_matrix):
    lead_shape = positions.shape
    flat = positions.reshape(-1)
    out = _build(flat.size)(flat, encoding_matrix)
    return out.reshape(*lead_shape, _D)
